# Initial kernel scaffold; baseline (speedup 1.0000x reference)
import jax
import jax.numpy as jnp
from jax.experimental import pallas as pl


def kernel(loc_preds, cls_preds, loc_targets, cls_targets):
    raise NotImplementedError("write your pallas kernel here")


# speedup vs baseline: 1.1731x; 1.1731x over previous
import functools

import jax
import jax.numpy as jnp
from jax import lax
from jax.experimental import pallas as pl
from jax.experimental.pallas import tpu as pltpu
from jax.experimental.pallas import tpu_sc as plsc

_B, _A, _C = 32, 8732, 81
_N = _B * _A
_G = 128
_R = _N // _G
_BR = 37
_S = _R // _BR
_LR = (_BR * _G * 4) // _G
_AP = 8832
_NSL = _AP // 16


def _tc_body(cls_ref, tgt_ref, locp_ref, loct_ref, mask_ref,
             ce_ref, loc_ref, acc_ref):
    x = cls_ref[...]
    t = tgt_ref[0]
    m = jnp.max(x, axis=2)
    e = jnp.exp(x - m[:, :, None])
    lse = m + jnp.log(jnp.sum(e, axis=2))
    ids = lax.broadcasted_iota(jnp.int32, (_BR, _G, _C), 2)
    picked = jnp.sum(jnp.where(ids == t[:, :, None], x, 0.0), axis=2)
    ce_ref[0] = lse - picked

    d = locp_ref[0] - loct_ref[0]
    ad = jnp.abs(d)
    sl1 = jnp.where(ad < 1.0, 0.5 * d * d, ad - 0.5)
    blk = jnp.sum(sl1 * mask_ref[0])
    i = pl.program_id(0)
    tot = jnp.where(i == 0, blk, acc_ref[0, 0] + blk)
    acc_ref[0, 0] = tot

    @pl.when(i == _S - 1)
    def _():
        loc_ref[0, 0] = tot


def _tc_pass(cls3, tgt3, locp3, loct3, mask3):
    return pl.pallas_call(
        _tc_body,
        grid=(_S,),
        in_specs=[
            pl.BlockSpec((_BR, _G, _C), lambda i: (i, 0, 0)),
            pl.BlockSpec((1, _BR, _G), lambda i: (i, 0, 0)),
            pl.BlockSpec((1, _LR, _G), lambda i: (i, 0, 0)),
            pl.BlockSpec((1, _LR, _G), lambda i: (i, 0, 0)),
            pl.BlockSpec((1, _LR, _G), lambda i: (i, 0, 0)),
        ],
        out_specs=[
            pl.BlockSpec((1, _BR, _G), lambda i: (i, 0, 0)),
            pl.BlockSpec(memory_space=pltpu.SMEM),
        ],
        out_shape=[
            jax.ShapeDtypeStruct((_S, _BR, _G), jnp.float32),
            jax.ShapeDtypeStruct((1, 1), jnp.float32),
        ],
        scratch_shapes=[pltpu.SMEM((1, 1), jnp.float32)],
    )(cls3, tgt3, locp3, loct3, mask3)


def _sc_mine(ce_flat, tgt_flat):
    mesh = plsc.VectorSubcoreMesh(core_axis_name="c", subcore_axis_name="s")

    @functools.partial(
        pl.kernel,
        mesh=mesh,
        out_type=[
            jax.ShapeDtypeStruct((_B * 16,), jnp.int32),
            jax.ShapeDtypeStruct((_B * 16,), jnp.float32),
            jax.ShapeDtypeStruct((_B * 16,), jnp.float32),
        ],
        scratch_types=[
            pltpu.VMEM((_AP,), jnp.float32),
            pltpu.VMEM((_AP,), jnp.int32),
            pltpu.VMEM((16,), jnp.int32),
            pltpu.VMEM((16,), jnp.float32),
            pltpu.VMEM((16,), jnp.float32),
        ],
    )
    def mine(ce_hbm, tgt_hbm, npo_hbm, spo_hbm, sno_hbm,
             ce_v, tgt_v, oa_v, ob_v, oc_v):
        row = lax.axis_index("c") * 16 + lax.axis_index("s")
        pltpu.sync_copy(ce_hbm.at[pl.ds(row * _AP, _AP)], ce_v)
        pltpu.sync_copy(tgt_hbm.at[pl.ds(row * _AP, _AP)], tgt_v)
        zi = jnp.zeros((16,), jnp.int32)
        zf = jnp.zeros((16,), jnp.float32)
        onei = jnp.ones((16,), jnp.int32)

        def p1(i, carry):
            npos, spos, sneg = carry
            v = ce_v[pl.ds(i * 16, 16)]
            t = tgt_v[pl.ds(i * 16, 16)]
            isp = t > 0
            return (npos + jnp.where(isp, onei, zi),
                    spos + jnp.where(isp, v, 0.0),
                    sneg + jnp.where(isp, 0.0, v))

        npv, spv, snv = lax.fori_loop(0, _NSL, p1, (zi, zf, zf))
        oa_v[...] = npv
        ob_v[...] = spv
        oc_v[...] = snv
        pltpu.sync_copy(oa_v, npo_hbm.at[pl.ds(row * 16, 16)])
        pltpu.sync_copy(ob_v, spo_hbm.at[pl.ds(row * 16, 16)])
        pltpu.sync_copy(oc_v, sno_hbm.at[pl.ds(row * 16, 16)])

    return mine(ce_flat, tgt_flat)


def _rare_body(ce_ref, tgt_ref, np_ref, loc_ref, out_ref):
    ce = ce_ref[...]
    t = tgt_ref[...]
    np_b = np_ref[...].astype(jnp.float32)
    k = 3.0 * np_b
    nneg = float(_A) - np_b
    isp = t > 0
    spos_b = jnp.sum(jnp.where(isp, ce, 0.0), axis=1, keepdims=True)
    sneg_b = jnp.sum(jnp.where(isp, 0.0, ce), axis=1, keepdims=True)
    u = jnp.where(isp, jnp.int32(-1), lax.bitcast_convert_type(ce, jnp.int32))
    ki = (3 * np_ref[...]).astype(jnp.int32)

    def sbit(j, thr):
        cand = thr | jnp.left_shift(jnp.int32(1), 30 - j)
        cnt = jnp.sum((u >= cand).astype(jnp.int32), axis=1, keepdims=True)
        return jnp.where(cnt >= ki, cand, thr)

    thr = lax.fori_loop(0, 31, sbit, jnp.zeros((_B, 1), jnp.int32))
    gt = u > thr
    cnt_gt = jnp.sum(gt.astype(jnp.int32), axis=1, keepdims=True)
    sum_gt = jnp.sum(jnp.where(gt, ce, 0.0), axis=1, keepdims=True)
    tval = lax.bitcast_convert_type(thr, jnp.float32)
    sel_rare = sum_gt + (ki - cnt_gt).astype(jnp.float32) * tval
    sel_rare = jnp.where(ki == 0, 0.0, sel_rare)
    sel = spos_b + jnp.where(k >= nneg, sneg_b, sel_rare)
    num_pos = jnp.sum(np_b)
    out_ref[0, 0] = (loc_ref[0, 0] + jnp.sum(sel)) / num_pos


def _rare_pass(ce_pad, tgt_pad, np_b, loc_sum):
    return pl.pallas_call(
        _rare_body,
        in_specs=[
            pl.BlockSpec((_B, _AP), lambda: (0, 0)),
            pl.BlockSpec((_B, _AP), lambda: (0, 0)),
            pl.BlockSpec((_B, 1), lambda: (0, 0)),
            pl.BlockSpec(memory_space=pltpu.SMEM),
        ],
        out_specs=pl.BlockSpec(memory_space=pltpu.SMEM),
        out_shape=jax.ShapeDtypeStruct((1, 1), jnp.float32),
    )(ce_pad, tgt_pad, np_b, loc_sum)


def kernel(loc_preds, cls_preds, loc_targets, cls_targets):
    tgt = cls_targets.astype(jnp.int32)
    cls3 = cls_preds.reshape(_R, _G, _C)
    tgt3 = tgt.reshape(_S, _BR, _G)
    posrep = jnp.repeat(
        (tgt.reshape(-1) > 0).astype(jnp.float32), 4).reshape(_S, _LR, _G)
    locp3 = loc_preds.reshape(_S, _LR, _G)
    loct3 = loc_targets.reshape(_S, _LR, _G)
    ce3, loc_sum = _tc_pass(cls3, tgt3, locp3, loct3, posrep)
    ce_pad = jnp.pad(ce3.reshape(_B, _A), ((0, 0), (0, _AP - _A)))
    tgt_pad = jnp.pad(tgt, ((0, 0), (0, _AP - _A)))
    npo, spo, sno = _sc_mine(ce_pad.reshape(-1), tgt_pad.reshape(-1))
    np_b = jnp.sum(npo.reshape(_B, 16), axis=1)
    sel_fast = jnp.sum(spo) + jnp.sum(sno)
    num_pos = jnp.sum(np_b).astype(jnp.float32)
    loss_fast = (loc_sum[0, 0] + sel_fast) / num_pos
    any_rare = jnp.any(4 * np_b < _A)
    return lax.cond(
        any_rare,
        lambda: _rare_pass(ce_pad, tgt_pad, np_b[:, None], loc_sum)[0, 0],
        lambda: loss_fast,
    )
